# baseline (speedup 1.0000x reference)
import functools

import jax
import jax.numpy as jnp
from jax import lax
from jax.experimental import pallas as pl
from jax.experimental.pallas import tpu as pltpu
from jax.experimental.pallas import tpu_sc as plsc

D = 32
L = 16
NW = 32
C = 128


def _embed_lookup(idx_flat, e1, e2, half):
    n = idx_flat.shape[0]
    per_w = n // NW
    n_chunks = per_w // C

    mesh = plsc.VectorSubcoreMesh(core_axis_name="c", subcore_axis_name="s")

    @functools.partial(
        pl.kernel,
        mesh=mesh,
        compiler_params=pltpu.CompilerParams(use_tc_tiling_on_sc=False),
        out_type=jax.ShapeDtypeStruct((n, D), jnp.float32),
        scratch_types=[
            pltpu.VMEM((C,), jnp.int32),
            pltpu.VMEM((C,), jnp.int32),
            pltpu.VMEM((C,), jnp.int32),
            pltpu.VMEM((C, D), jnp.float32),
            pltpu.VMEM((C, D), jnp.float32),
            pltpu.SemaphoreType.DMA,
            pltpu.SemaphoreType.DMA,
        ],
    )
    def k(idx_hbm, e1_hbm, e2_hbm, out_hbm, idx_v, i1_v, i2_v, b1, b2, s1, s2):
        wid = lax.axis_index("s") * 2 + lax.axis_index("c")
        base_w = wid * per_w

        def chunk(ci, carry):
            base = base_w + ci * C
            pltpu.sync_copy(idx_hbm.at[pl.ds(base, C)], idx_v)
            for g in range(C // L):
                v = idx_v[pl.ds(g * L, L)]
                i1_v[pl.ds(g * L, L)] = jnp.minimum(v, half - 1)
                i2_v[pl.ds(g * L, L)] = jnp.maximum(v - half, 0)
            cp1 = pltpu.async_copy(e1_hbm.at[i1_v], b1, s1)
            cp2 = pltpu.async_copy(e2_hbm.at[i2_v], b2, s2)
            cp1.wait()
            cp2.wait()

            def row_group(g, c2):
                iv = idx_v[pl.ds(g * L, L)]
                for j in range(L):
                    r = g * L + j
                    take_e1 = iv[j] < half
                    lo = jnp.where(take_e1, b1[r, pl.ds(0, L)], b2[r, pl.ds(0, L)])
                    hi = jnp.where(take_e1, b1[r, pl.ds(L, L)], b2[r, pl.ds(L, L)])
                    b1[r, pl.ds(0, L)] = lo
                    b1[r, pl.ds(L, L)] = hi
                return c2

            lax.fori_loop(0, C // L, row_group, 0)
            pltpu.sync_copy(b1, out_hbm.at[pl.ds(base, C)])
            return carry

        lax.fori_loop(0, n_chunks, chunk, 0)

    return k(idx_flat, e1, e2)


def kernel(inputs, e1, e2):
    b, h = inputs.shape
    half = e1.shape[0]
    idx_flat = inputs.reshape(b * h).astype(jnp.int32)
    out = _embed_lookup(idx_flat, e1, e2, half)
    return out.reshape(b, h, D)
